# baseline (speedup 1.0000x reference)
import functools

import jax
import jax.numpy as jnp
from jax import lax
from jax.experimental import pallas as pl
from jax.experimental.pallas import tpu as pltpu
from jax.experimental.pallas import tpu_sc as plsc

B = 16384
D = 32
PK = 4
DW = D * PK
NC = 2
NS = 16
NW = NC * NS
BPW = B // NW
CH = 128
NCH = BPW // CH

BLK = 2048


def _gather_body(ut, it, uid3, iid3, u_out, i_out, idx, rows, sem):
    wid = lax.axis_index("s") * NC + lax.axis_index("c")
    base = wid * BPW
    for table, ids, out in ((ut, uid3, u_out), (it, iid3, i_out)):
        pltpu.sync_copy(ids.at[wid], idx)
        copies = [
            pltpu.async_copy(table.at[idx.at[j]], rows.at[j], sem)
            for j in range(NCH)
        ]
        for c in copies:
            c.wait()
        for j in range(NCH):
            pltpu.sync_copy(rows.at[j], out.at[pl.ds(base + j * CH, CH)])


def _sc_gather(ut4, it4, uid3, iid3):
    mesh = plsc.VectorSubcoreMesh(core_axis_name="c", subcore_axis_name="s")
    fn = functools.partial(
        pl.kernel,
        mesh=mesh,
        out_type=(
            jax.ShapeDtypeStruct((B, DW), jnp.float32),
            jax.ShapeDtypeStruct((B, DW), jnp.float32),
        ),
        scratch_types=[
            pltpu.VMEM((NCH, CH), jnp.int32),
            pltpu.VMEM((NCH, CH, DW), jnp.float32),
            pltpu.SemaphoreType.DMA,
        ],
    )(_gather_body)
    return fn(ut4, it4, uid3, iid3)


def _mlp_body(u, i, ulo, ilo, w1, b1, w2, b2, w3t, b3, o):
    uraw = u[...]
    iraw = i[...]
    ulo_c = ulo[...].reshape(BLK, 1)
    ilo_c = ilo[...].reshape(BLK, 1)
    ue = uraw[:, 0:D]
    ie = iraw[:, 0:D]
    for k in range(1, PK):
        ue = jnp.where(ulo_c == k, uraw[:, k * D:(k + 1) * D], ue)
        ie = jnp.where(ilo_c == k, iraw[:, k * D:(k + 1) * D], ie)
    f = jnp.concatenate([ue, ie], axis=1)
    h = jnp.maximum(
        jnp.dot(f, w1[...], preferred_element_type=jnp.float32) + b1[...], 0.0)
    h = jnp.maximum(
        jnp.dot(h, w2[...], preferred_element_type=jnp.float32) + b2[...], 0.0)
    z = jnp.sum(h * w3t[...], axis=1) + b3[0, 0]
    o[...] = jax.nn.sigmoid(z)


def _tc_mlp(u_raw, i_raw, u_lo, i_lo, W1, b1, W2, b2, W3, b3):
    b1r = b1.reshape(1, -1)
    b2r = b2.reshape(1, -1)
    w3t = W3.reshape(1, -1)
    b3r = b3.reshape(1, 1)
    grid = (B // BLK,)
    return pl.pallas_call(
        _mlp_body,
        grid=grid,
        in_specs=[
            pl.BlockSpec((BLK, DW), lambda idx: (idx, 0)),
            pl.BlockSpec((BLK, DW), lambda idx: (idx, 0)),
            pl.BlockSpec((BLK,), lambda idx: (idx,)),
            pl.BlockSpec((BLK,), lambda idx: (idx,)),
            pl.BlockSpec(W1.shape, lambda idx: (0, 0)),
            pl.BlockSpec(b1r.shape, lambda idx: (0, 0)),
            pl.BlockSpec(W2.shape, lambda idx: (0, 0)),
            pl.BlockSpec(b2r.shape, lambda idx: (0, 0)),
            pl.BlockSpec(w3t.shape, lambda idx: (0, 0)),
            pl.BlockSpec(memory_space=pltpu.SMEM),
        ],
        out_specs=pl.BlockSpec((BLK,), lambda idx: (idx,)),
        out_shape=jax.ShapeDtypeStruct((B,), jnp.float32),
    )(u_raw, i_raw, u_lo, i_lo, W1, b1r, W2, b2r, w3t, b3r)


def kernel(user_table, item_table, W1, b1, W2, b2, W3, b3, user_ids, item_ids):
    uids = user_ids.astype(jnp.int32)
    iids = item_ids.astype(jnp.int32)
    ut4 = user_table.reshape(-1, DW)
    it4 = item_table.reshape(-1, DW)
    uid3 = (uids // PK).reshape(NW, NCH, CH)
    iid3 = (iids // PK).reshape(NW, NCH, CH)
    u_lo = uids % PK
    i_lo = iids % PK
    u_raw, i_raw = _sc_gather(ut4, it4, uid3, iid3)
    return _tc_mlp(u_raw, i_raw, u_lo, i_lo, W1, b1, W2, b2, W3, b3)
